# Initial kernel scaffold; baseline (speedup 1.0000x reference)
import jax
import jax.numpy as jnp
from jax.experimental import pallas as pl


def kernel(att_feat, aud_feat, att_heatmaps, av_heatmaps):
    raise NotImplementedError("write your pallas kernel here")


# speedup vs baseline: 4.4804x; 4.4804x over previous
import jax
import jax.numpy as jnp
from jax.experimental import pallas as pl
from jax.experimental.pallas import tpu as pltpu

FG = 128
BG = 128
_B, _C, _K = 64, 512, 1024
_P = 1024
_I32_MIN = -2147483648
_M31 = 2147483647


def _sim_kernel(att_ref, aud_ref, sim_ref):
    a = att_ref[0]
    aud = aud_ref[0]
    dot = jnp.dot(aud, a, preferred_element_type=jnp.float32)
    nsq = jnp.sum(a * a, axis=0, keepdims=True)
    sim_ref[0] = dot / jnp.maximum(jnp.sqrt(nsq), 1e-12)


def _key(x_i32):
    return jnp.where(x_i32 < 0, x_i32 ^ jnp.int32(_M31), x_i32)


def _select_kernel(sim_ref, dis_ref, w_ref):
    sim = sim_ref[...]
    ka = _key(jax.lax.bitcast_convert_type(sim, jnp.int32))

    def body(j, P):
        bit = 31 - j
        phi, plo = P
        step = jnp.int32(1) << bit
        chi = phi + step
        clo = plo + step
        cnt_hi = jnp.sum((ka >= chi).astype(jnp.int32), axis=1, keepdims=True)
        cnt_lo = jnp.sum((ka >= clo).astype(jnp.int32), axis=1, keepdims=True)
        phi = jnp.where(cnt_hi >= FG, chi, phi)
        plo = jnp.where(cnt_lo >= _K - BG + 1, clo, plo)
        return (phi, plo)

    p0 = jnp.full((_B, 1), _I32_MIN, jnp.int32)
    phi, plo = jax.lax.fori_loop(0, 32, body, (p0, p0))

    thi_f = jax.lax.bitcast_convert_type(_key(phi), jnp.float32)
    tlo_f = jax.lax.bitcast_convert_type(_key(plo), jnp.float32)

    gt = ka > phi
    cnt_gt = jnp.sum(gt.astype(jnp.float32), axis=1, keepdims=True)
    sum_gt = jnp.sum(jnp.where(gt, sim, 0.0), axis=1, keepdims=True)
    pos = (sum_gt + thi_f * (FG - cnt_gt)) * (1.0 / FG)

    lt = ka < plo
    cnt_lt = jnp.sum(lt.astype(jnp.float32), axis=1, keepdims=True)
    sum_lt = jnp.sum(jnp.where(lt, sim, 0.0), axis=1, keepdims=True)
    hn = (sum_lt + tlo_f * (BG - cnt_lt)) * (1.0 / BG)

    m = jnp.maximum(pos, hn)
    logz = m + jnp.log(jnp.exp(pos - m) + jnp.exp(hn - m))
    dis = jnp.mean(logz - pos)
    dis_ref[...] = jnp.full((8, 128), dis, jnp.float32)

    eq = (ka == phi)
    row = jax.lax.broadcasted_iota(jnp.int32, (_K, _K), 0)
    col = jax.lax.broadcasted_iota(jnp.int32, (_K, _K), 1)
    tri = (row <= col).astype(jnp.float32)
    cum_eq = jax.lax.dot_general(
        eq.astype(jnp.float32), tri, (((1,), (0,)), ((), ())),
        precision=jax.lax.Precision.HIGHEST)
    r = FG - cnt_gt
    sel = gt | (eq & (cum_eq <= r + 0.5))
    w_ref[...] = sel.astype(jnp.float32) * (1.0 / FG)


def _combine_kernel(w_ref, hm_ref, av_ref, acc_ref):
    b = pl.program_id(0)
    w = w_ref[0]
    h = hm_ref[0]
    comb = jnp.dot(w, h, preferred_element_type=jnp.float32,
                   precision=jax.lax.Precision.HIGHEST)

    cmax = jnp.max(comb, axis=1, keepdims=True)
    ce = jnp.exp(comb - cmax)
    att = ce / jnp.sum(ce, axis=1, keepdims=True)

    av = av_ref[0]
    vmax = jnp.max(av, axis=1, keepdims=True)
    ve = jnp.exp(av - vmax)
    avd = ve / jnp.sum(ve, axis=1, keepdims=True)

    lm = jnp.log((att + avd) * 0.5)
    term = (jnp.sum(att * (jnp.log(att) - lm)) +
            jnp.sum(avd * (jnp.log(avd) - lm)))

    @pl.when(b == 0)
    def _():
        acc_ref[...] = jnp.zeros_like(acc_ref)

    acc_ref[...] += jnp.full((8, 128), term, jnp.float32)


def kernel(att_feat, aud_feat, att_heatmaps, av_heatmaps):
    B, C, K = att_feat.shape
    P = att_heatmaps.shape[2] * att_heatmaps.shape[3]

    sim = pl.pallas_call(
        _sim_kernel,
        grid=(B,),
        in_specs=[
            pl.BlockSpec((1, C, K), lambda b: (b, 0, 0)),
            pl.BlockSpec((1, 1, C), lambda b: (b, 0, 0)),
        ],
        out_specs=pl.BlockSpec((1, 1, K), lambda b: (b, 0, 0)),
        out_shape=jax.ShapeDtypeStruct((B, 1, K), jnp.float32),
    )(att_feat, aud_feat.reshape(B, 1, C)).reshape(B, K)

    dis, w = pl.pallas_call(
        _select_kernel,
        in_specs=[pl.BlockSpec((B, K), lambda: (0, 0))],
        out_specs=[
            pl.BlockSpec((8, 128), lambda: (0, 0)),
            pl.BlockSpec((B, K), lambda: (0, 0)),
        ],
        out_shape=[
            jax.ShapeDtypeStruct((8, 128), jnp.float32),
            jax.ShapeDtypeStruct((B, K), jnp.float32),
        ],
    )(sim)

    acc = pl.pallas_call(
        _combine_kernel,
        grid=(B,),
        in_specs=[
            pl.BlockSpec((1, 1, K), lambda b: (b, 0, 0)),
            pl.BlockSpec((1, K, P), lambda b: (b, 0, 0)),
            pl.BlockSpec((1, 1, P), lambda b: (b, 0, 0)),
        ],
        out_specs=pl.BlockSpec((8, 128), lambda b: (0, 0)),
        out_shape=jax.ShapeDtypeStruct((8, 128), jnp.float32),
    )(w.reshape(B, 1, K), att_heatmaps.reshape(B, K, P),
      av_heatmaps.reshape(B, 1, P))

    dis_loss = dis[0, 0].reshape(())
    div_loss = (acc[0, 0] / (2.0 * B)).reshape(())
    return dis_loss, div_loss
